# baseline (speedup 1.0000x reference)
import jax
import jax.numpy as jnp
from jax import lax
from jax.experimental import pallas as pl
from jax.experimental.pallas import tpu as pltpu
from jax.experimental.pallas import tpu_sc as plsc

NG = 8


def _colbc(row):
    return jnp.swapaxes(jnp.broadcast_to(row, (128, 128)), 0, 1)


def _tc_prep(degA, degB, t2d, hchunks, W, Cn):
    C = len(hchunks)
    F = C * 128
    NT = degA.shape[0]

    def body(*refs):
        dA, dB, t, w_ref = refs[0], refs[1], refs[2], refs[3]
        hs = refs[4:4 + C]
        ys = refs[4 + C:4 + C + Cn]
        dis_ref = refs[4 + C + Cn]
        deg = (dA[...] + dB[...]).reshape(1, 128)
        dis = jnp.where(deg > 0.0, lax.rsqrt(jnp.maximum(deg, 1.0)), 0.0)
        Mt = _colbc(t[...].reshape(1, 128))
        xt = jnp.concatenate([hs[c][...] * Mt for c in range(C)], axis=1)
        g = jnp.dot(xt, w_ref[...], preferred_element_type=jnp.float32)
        Md = _colbc(dis)
        for c in range(Cn):
            ys[c][...] = g[:, c * 128:(c + 1) * 128] * Md
        dis_ref[...] = dis.reshape(1, 1, 128)

    row = pl.BlockSpec((1, 1, 128), lambda i: (i, 0, 0))
    blk = pl.BlockSpec((128, 128), lambda i: (i, 0))
    outs = pl.pallas_call(
        body,
        grid=(NT,),
        in_specs=[row, row, row,
                  pl.BlockSpec((F, Cn * 128), lambda i: (0, 0))] + [blk] * C,
        out_specs=[blk] * Cn + [row],
        out_shape=[jax.ShapeDtypeStruct((NT * 128, 128), jnp.float32)] * Cn
        + [jax.ShapeDtypeStruct((NT, 1, 128), jnp.float32)],
    )(degA, degB, t2d, W, *hchunks)
    return outs[:Cn], outs[Cn]


def _tc_hpost(dis2d, brow, agg, n_real):
    C = len(agg)
    NT = dis2d.shape[0]
    F = C * 128

    def body(*refs):
        dis_ref, b_ref = refs[0], refs[1]
        aggs = refs[2:2 + C]
        hs = refs[2 + C:]
        i = pl.program_id(0)
        Md = _colbc(dis_ref[...].reshape(1, 128))
        ri = lax.broadcasted_iota(jnp.int32, (128, 128), 0)
        rowmask = (i * 128 + ri) < n_real
        b = b_ref[...]
        for c in range(C):
            z = aggs[c][...] * Md + b[:, c * 128:(c + 1) * 128]
            h = jnp.where(z >= 0.0, z, 0.01 * z)
            hs[c][...] = jnp.where(rowmask, h, 0.0)

    row = pl.BlockSpec((1, 1, 128), lambda i: (i, 0, 0))
    blk = pl.BlockSpec((128, 128), lambda i: (i, 0))
    return pl.pallas_call(
        body,
        grid=(NT,),
        in_specs=[row, pl.BlockSpec((1, F), lambda i: (0, 0))] + [blk] * C,
        out_specs=[blk] * C,
        out_shape=[jax.ShapeDtypeStruct((NT * 128, 128), jnp.float32)] * C,
    )(dis2d, brow, *agg)


def _tc_score(sagg, h, Pstack):
    C = len(sagg)
    F2 = 2 * C * 128
    NT = sagg[0].shape[0] // 128

    def body(*refs):
        p_ref = refs[0]
        tiles = refs[1:1 + 2 * C]
        score_ref = refs[1 + 2 * C]
        tile = jnp.concatenate([r[...] for r in tiles], axis=1)
        g = jnp.dot(tile, p_ref[...], preferred_element_type=jnp.float32)
        tr = jnp.swapaxes(g, 0, 1)
        score_ref[...] = tr[0:1, :].reshape(1, 1, 128)

    blk = pl.BlockSpec((128, 128), lambda i: (i, 0))
    return pl.pallas_call(
        body,
        grid=(NT,),
        in_specs=[pl.BlockSpec((F2, 128), lambda i: (0, 0))] + [blk] * (2 * C),
        out_specs=pl.BlockSpec((1, 1, 128), lambda i: (i, 0, 0)),
        out_shape=jax.ShapeDtypeStruct((NT, 1, 128), jnp.float32),
    )(Pstack, *sagg, *h)


def _tc_select(num, den, score2d, brel, batchf, act2d):
    NT = score2d.shape[0]
    NP = NT * 128
    NBITS = max(1, (NP - 1).bit_length())

    def body(sc_ref, br, bf_ref, act_ref, keep_ref, t_ref):
        score = sc_ref[...].reshape(NT, 128) + br[...]
        sbits = lax.bitcast_convert_type(score, jnp.int32)
        ubits = lax.bitcast_convert_type(score, jnp.uint32)
        key = jnp.where(sbits < 0, ~ubits, ubits | jnp.uint32(0x80000000))
        gi = (lax.broadcasted_iota(jnp.int32, (NT, 128), 0) * 128
              + lax.broadcasted_iota(jnp.int32, (NT, 128), 1))
        actm = act_ref[...].reshape(NT, 128) > 0.0
        bf = bf_ref[...].reshape(NT, 128)
        keep = jnp.zeros((NT, 128), jnp.float32)
        for g in range(NG):
            m = jnp.logical_and(bf == float(g), actm)
            c = jnp.sum(m.astype(jnp.int32))
            k = (num * c + (den - 1)) // den

            def tbody(it, T):
                bit = lax.shift_left(jnp.uint32(1), (31 - it).astype(jnp.uint32))
                Tc = T | bit
                cnt = jnp.sum(jnp.where(jnp.logical_and(m, key >= Tc), 1, 0))
                return jnp.where(cnt >= k, Tc, T)

            T = lax.fori_loop(0, 32, tbody, jnp.uint32(0))
            cgt = jnp.sum(jnp.where(jnp.logical_and(m, key > T), 1, 0))
            q = k - cgt
            meq = jnp.logical_and(m, key == T)

            def ebody(it, Ei):
                bit = lax.shift_left(jnp.int32(1), (NBITS - 1 - it).astype(jnp.int32))
                Ec = Ei | bit
                ce = jnp.sum(jnp.where(jnp.logical_and(meq, gi < Ec), 1, 0))
                return jnp.where(ce < q, Ec, Ei)

            E = lax.fori_loop(0, NBITS, ebody, jnp.int32(0))
            kg = jnp.logical_and(
                m,
                jnp.logical_or(
                    key > T,
                    jnp.logical_and(jnp.logical_and(meq, gi <= E), q > 0)))
            keep = keep + kg.astype(jnp.float32)
        keep_ref[...] = keep.reshape(NT, 1, 128)
        t_ref[...] = (jnp.tanh(score) * keep).reshape(NT, 1, 128)

    return pl.pallas_call(
        body,
        out_shape=[jax.ShapeDtypeStruct((NT, 1, 128), jnp.float32)] * 2,
    )(score2d, brel, batchf, act2d)


def _tc_final(dis2d, b4row, batchf, keepf, Wlp, blr, a0, a1):
    NT = dis2d.shape[0]
    F = 256

    def body(dis_ref, b_ref, bf_ref, k_ref, w_ref, bl_ref, a0_ref, a1_ref,
             out_ref, maxs, sums, cnts):
        i = pl.program_id(0)

        @pl.when(i == 0)
        def _init():
            maxs[...] = jnp.full((NG, F), -jnp.inf, jnp.float32)
            sums[...] = jnp.zeros((NG, F), jnp.float32)
            cnts[...] = jnp.zeros((NG, 128), jnp.float32)

        Md = _colbc(dis_ref[...].reshape(1, 128))
        tile = jnp.concatenate([a0_ref[...] * Md, a1_ref[...] * Md], axis=1) + b_ref[...]
        bcol = _colbc(bf_ref[...].reshape(1, 128))[:, 0:1]
        kcol = _colbc(k_ref[...].reshape(1, 128))[:, 0:1]
        for g in range(NG):
            m = jnp.logical_and(bcol == float(g), kcol > 0.0)
            mv = jnp.where(m, tile, -jnp.inf)
            maxs[g:g + 1, :] = jnp.maximum(maxs[g:g + 1, :],
                                           jnp.max(mv, axis=0, keepdims=True))
            sv = jnp.where(m, tile, 0.0)
            sums[g:g + 1, :] = sums[g:g + 1, :] + jnp.sum(sv, axis=0, keepdims=True)
            cnts[g:g + 1, :] = cnts[g:g + 1, :] + jnp.sum(m.astype(jnp.float32))

        @pl.when(i == NT - 1)
        def _fin():
            cnt = cnts[...][:, 0:1]
            meanp = sums[...] / jnp.maximum(cnt, 1.0)
            feat = jnp.concatenate([maxs[...], meanp], axis=1)
            res = jnp.dot(feat, w_ref[...], preferred_element_type=jnp.float32)
            out_ref[...] = res[:, 0:1] + bl_ref[...]

    row = pl.BlockSpec((1, 1, 128), lambda i: (i, 0, 0))
    blk = pl.BlockSpec((128, 128), lambda i: (i, 0))
    return pl.pallas_call(
        body,
        grid=(NT,),
        in_specs=[row,
                  pl.BlockSpec((1, F), lambda i: (0, 0)),
                  row, row,
                  pl.BlockSpec((2 * F, 128), lambda i: (0, 0)),
                  pl.BlockSpec((1, 1), lambda i: (0, 0)),
                  blk, blk],
        out_specs=pl.BlockSpec((NG, 1), lambda i: (0, 0)),
        out_shape=jax.ShapeDtypeStruct((NG, 1), jnp.float32),
        scratch_shapes=[pltpu.VMEM((NG, F), jnp.float32),
                        pltpu.VMEM((NG, F), jnp.float32),
                        pltpu.VMEM((NG, 128), jnp.float32)],
    )(dis2d, b4row, batchf, keepf, Wlp, blr, a0, a1)


_SC_MESH = dict(core_axis_name="c", subcore_axis_name="s")
_SC_PARAMS = pltpu.CompilerParams(use_tc_tiling_on_sc=False,
                                  needs_layout_passes=False)


def _sc_edge_update(srcr, dstr, keepflat, zrow, n_zero):
    ER, _ = srcr.shape
    NP = keepflat.shape[0]
    SL = NP // 16
    RW = ER // 32
    NB = RW // 4
    mesh = plsc.VectorSubcoreMesh(**_SC_MESH)

    def body(src_h, dst_h, keep_h, z_h, newsrc_h, deg_h,
             keep_vm, si, di, vv, ns, acc):
        cid = lax.axis_index("c")
        sid = lax.axis_index("s")
        wid = sid * 2 + cid
        pltpu.sync_copy(z_h, acc.at[pl.ds(sid * SL, SL)])
        pltpu.sync_copy(keep_h, keep_vm)
        plsc.subcore_barrier()
        rbase = wid * RW

        def batch(b, carry):
            r0 = rbase + b * 4
            pltpu.sync_copy(src_h.at[pl.ds(r0, 4)], si)
            pltpu.sync_copy(dst_h.at[pl.ds(r0, 4)], di)
            for j in range(4):
                for w in range(8):
                    sl = pl.ds(w * 16, 16)
                    sv = si[j, sl]
                    dv = di[j, sl]
                    k1 = plsc.load_gather(keep_vm, [sv])
                    k2 = plsc.load_gather(keep_vm, [dv])
                    val = k1 * k2
                    vv[j, sl] = val
                    ns[j, sl] = jnp.where(val > 0.0, sv,
                                          jnp.full((16,), n_zero, jnp.int32))
            pltpu.sync_copy(ns, newsrc_h.at[pl.ds(r0, 4)])
            for j in range(4):
                pltpu.sync_copy(vv.at[j], acc.at[di.at[j]], add=True)
            return carry

        lax.fori_loop(0, NB, batch, 0)
        plsc.subcore_barrier()
        pltpu.sync_copy(acc.at[pl.ds(sid * SL, SL)],
                        deg_h.at[pl.ds(cid * NP + sid * SL, SL)])

    f = pl.kernel(
        body,
        out_type=[jax.ShapeDtypeStruct((ER, 128), jnp.int32),
                  jax.ShapeDtypeStruct((2 * NP,), jnp.float32)],
        mesh=mesh,
        compiler_params=_SC_PARAMS,
        scratch_types=[
            pltpu.VMEM((NP,), jnp.float32),
            pltpu.VMEM((4, 128), jnp.int32),
            pltpu.VMEM((4, 128), jnp.int32),
            pltpu.VMEM((4, 128), jnp.float32),
            pltpu.VMEM((4, 128), jnp.int32),
            pltpu.VMEM_SHARED((NP,), jnp.float32),
        ],
    )
    return f(srcr, dstr, keepflat, zrow)


def _sc_row_scatter(ychunks, srcr, dstr, zrows):
    C = len(ychunks)
    ER, _ = srcr.shape
    NP = ychunks[0].shape[0]
    SLR = NP // 16
    RW = ER // 16
    NB = RW // 2
    mesh = plsc.VectorSubcoreMesh(**_SC_MESH)

    def body(*refs):
        ys = refs[:C]
        src_h, dst_h, z_h = refs[C], refs[C + 1], refs[C + 2]
        aggs = refs[C + 3:C + 3 + C]
        si, di, rows, acc, sem = refs[C + 3 + C:]
        cid = lax.axis_index("c")
        sid = lax.axis_index("s")
        for chunk in range(C):
            @pl.when((chunk % 2) == cid)
            def _(chunk=chunk):
                pltpu.sync_copy(z_h, acc.at[pl.ds(sid * SLR, SLR)])
                plsc.subcore_barrier()
                rbase = sid * RW

                def batch(b, carry):
                    r0 = rbase + b * 2
                    pltpu.sync_copy(src_h.at[pl.ds(r0, 2)], si)
                    pltpu.sync_copy(dst_h.at[pl.ds(r0, 2)], di)
                    cp0 = pltpu.async_copy(ys[chunk].at[si.at[0]],
                                           rows.at[pl.ds(0, 128)], sem)
                    cp1 = pltpu.async_copy(ys[chunk].at[si.at[1]],
                                           rows.at[pl.ds(128, 128)], sem)
                    cp0.wait()
                    cp1.wait()
                    pltpu.sync_copy(rows.at[pl.ds(0, 128)],
                                    acc.at[di.at[0]], add=True)
                    pltpu.sync_copy(rows.at[pl.ds(128, 128)],
                                    acc.at[di.at[1]], add=True)
                    return carry

                lax.fori_loop(0, NB, batch, 0)
                plsc.subcore_barrier()
                pltpu.sync_copy(acc.at[pl.ds(sid * SLR, SLR)],
                                aggs[chunk].at[pl.ds(sid * SLR, SLR)])

    f = pl.kernel(
        body,
        out_type=[jax.ShapeDtypeStruct((NP, 128), jnp.float32)] * C,
        mesh=mesh,
        compiler_params=_SC_PARAMS,
        scratch_types=[
            pltpu.VMEM((2, 128), jnp.int32),
            pltpu.VMEM((2, 128), jnp.int32),
            pltpu.VMEM((256, 128), jnp.float32),
            pltpu.VMEM_SHARED((NP, 128), jnp.float32),
            pltpu.SemaphoreType.DMA,
        ],
    )
    outs = f(*ychunks, srcr, dstr, zrows)
    return list(outs) if isinstance(outs, (list, tuple)) else [outs]


def kernel(x, edge_index, batch, W1, b1, W2, b2, W3, b3, W4, b4,
           P1_rel, P1_root, P1_b, P2_rel, P2_root, P2_b,
           P3_rel, P3_root, P3_b, Wl, bl):
    N, F0 = x.shape
    E = edge_index.shape[1]
    NP = ((N + 1 + 2047) // 2048) * 2048
    NT = NP // 128
    EP = ((E + 16383) // 16384) * 16384

    f32 = jnp.float32

    xp = jnp.pad(x, ((0, NP - N), (0, 0)))
    xch = [xp[:, c * 128:(c + 1) * 128] for c in range(F0 // 128)]
    src0 = jnp.concatenate(
        [edge_index[0].astype(jnp.int32),
         jnp.full((EP - E,), N, jnp.int32)]).reshape(EP // 128, 128)
    dstr = jnp.concatenate(
        [edge_index[1].astype(jnp.int32),
         jnp.zeros((EP - E,), jnp.int32)]).reshape(EP // 128, 128)
    batchf = jnp.concatenate(
        [batch.astype(f32), jnp.full((NP - N,), float(NG), f32)]).reshape(NT, 1, 128)
    act0_flat = (jnp.arange(NP) < N).astype(f32)
    act0 = act0_flat.reshape(NT, 1, 128)
    ones2d = jnp.ones((NT, 1, 128), f32)
    zrow = jnp.zeros((NP // 16,), f32)
    zrows = jnp.zeros((NP // 16, 128), f32)

    def pstack(Pr, Pt):
        ps = jnp.concatenate([Pr, Pt], axis=0)
        return jnp.concatenate([ps, jnp.zeros((ps.shape[0], 127), f32)], axis=1)

    Pst1 = pstack(P1_rel, P1_root)
    Pst2 = pstack(P2_rel, P2_root)
    Pst3 = pstack(P3_rel, P3_root)
    Wlp = jnp.concatenate([Wl, jnp.zeros((Wl.shape[0], 127), f32)], axis=1)
    b1r, b2r = b1.reshape(1, -1), b2.reshape(1, -1)
    b3r, b4r = b3.reshape(1, -1), b4.reshape(1, -1)
    blr = bl.reshape(1, 1)

    def parts(flat):
        return flat[:NP].reshape(NT, 1, 128), flat[NP:].reshape(NT, 1, 128)

    src1, deg1 = _sc_edge_update(src0, dstr, act0_flat, zrow, N)
    y1, dis1 = _tc_prep(*parts(deg1), ones2d, xch, W1, 4)
    agg1 = _sc_row_scatter(y1, src1, dstr, zrows)
    h1 = _tc_hpost(dis1, b1r, agg1, N)
    sagg1 = _sc_row_scatter(h1, src1, dstr, zrows)
    score1 = _tc_score(sagg1, h1, Pst1)
    keep1, t1 = _tc_select(3, 5, score1, P1_b.reshape(1, 1), batchf, act0)

    src2, deg2 = _sc_edge_update(src1, dstr, keep1.reshape(NP), zrow, N)
    y2, dis2 = _tc_prep(*parts(deg2), t1, h1, W2, 4)
    agg2 = _sc_row_scatter(y2, src2, dstr, zrows)
    h2 = _tc_hpost(dis2, b2r, agg2, N)
    sagg2 = _sc_row_scatter(h2, src2, dstr, zrows)
    score2 = _tc_score(sagg2, h2, Pst2)
    keep2, t2 = _tc_select(3, 5, score2, P2_b.reshape(1, 1), batchf, keep1)

    src3, deg3 = _sc_edge_update(src2, dstr, keep2.reshape(NP), zrow, N)
    y3, dis3 = _tc_prep(*parts(deg3), t2, h2, W3, 2)
    agg3 = _sc_row_scatter(y3, src3, dstr, zrows)
    h3 = _tc_hpost(dis3, b3r, agg3, N)
    sagg3 = _sc_row_scatter(h3, src3, dstr, zrows)
    score3 = _tc_score(sagg3, h3, Pst3)
    keep3, t3 = _tc_select(1, 2, score3, P3_b.reshape(1, 1), batchf, keep2)

    src4, deg4 = _sc_edge_update(src3, dstr, keep3.reshape(NP), zrow, N)
    y4, dis4 = _tc_prep(*parts(deg4), t3, h3, W4, 2)
    agg4 = _sc_row_scatter(y4, src4, dstr, zrows)
    return _tc_final(dis4, b4r, batchf, keep3, Wlp, blr, agg4[0], agg4[1])


# speedup vs baseline: 1.0084x; 1.0084x over previous
import jax
import jax.numpy as jnp
from jax import lax
from jax.experimental import pallas as pl
from jax.experimental.pallas import tpu as pltpu
from jax.experimental.pallas import tpu_sc as plsc

NG = 8


def _colbc(row):
    return jnp.swapaxes(jnp.broadcast_to(row, (128, 128)), 0, 1)


def _tc_prep(degA, degB, t2d, hchunks, W, Cn):
    C = len(hchunks)
    F = C * 128
    NT = degA.shape[0]

    def body(*refs):
        dA, dB, t, w_ref = refs[0], refs[1], refs[2], refs[3]
        hs = refs[4:4 + C]
        ys = refs[4 + C:4 + C + Cn]
        dis_ref = refs[4 + C + Cn]
        deg = (dA[...] + dB[...]).reshape(1, 128)
        dis = jnp.where(deg > 0.0, lax.rsqrt(jnp.maximum(deg, 1.0)), 0.0)
        Mt = _colbc(t[...].reshape(1, 128))
        xt = jnp.concatenate([hs[c][...] * Mt for c in range(C)], axis=1)
        g = jnp.dot(xt, w_ref[...], preferred_element_type=jnp.float32)
        Md = _colbc(dis)
        for c in range(Cn):
            ys[c][...] = g[:, c * 128:(c + 1) * 128] * Md
        dis_ref[...] = dis.reshape(1, 1, 128)

    row = pl.BlockSpec((1, 1, 128), lambda i: (i, 0, 0))
    blk = pl.BlockSpec((128, 128), lambda i: (i, 0))
    outs = pl.pallas_call(
        body,
        grid=(NT,),
        in_specs=[row, row, row,
                  pl.BlockSpec((F, Cn * 128), lambda i: (0, 0))] + [blk] * C,
        out_specs=[blk] * Cn + [row],
        out_shape=[jax.ShapeDtypeStruct((NT * 128, 128), jnp.float32)] * Cn
        + [jax.ShapeDtypeStruct((NT, 1, 128), jnp.float32)],
    )(degA, degB, t2d, W, *hchunks)
    return outs[:Cn], outs[Cn]


def _tc_hpost(dis2d, brow, agg, n_real):
    C = len(agg)
    NT = dis2d.shape[0]
    F = C * 128

    def body(*refs):
        dis_ref, b_ref = refs[0], refs[1]
        aggs = refs[2:2 + C]
        hs = refs[2 + C:]
        i = pl.program_id(0)
        Md = _colbc(dis_ref[...].reshape(1, 128))
        ri = lax.broadcasted_iota(jnp.int32, (128, 128), 0)
        rowmask = (i * 128 + ri) < n_real
        b = b_ref[...]
        for c in range(C):
            z = aggs[c][...] * Md + b[:, c * 128:(c + 1) * 128]
            h = jnp.where(z >= 0.0, z, 0.01 * z)
            hs[c][...] = jnp.where(rowmask, h, 0.0)

    row = pl.BlockSpec((1, 1, 128), lambda i: (i, 0, 0))
    blk = pl.BlockSpec((128, 128), lambda i: (i, 0))
    return pl.pallas_call(
        body,
        grid=(NT,),
        in_specs=[row, pl.BlockSpec((1, F), lambda i: (0, 0))] + [blk] * C,
        out_specs=[blk] * C,
        out_shape=[jax.ShapeDtypeStruct((NT * 128, 128), jnp.float32)] * C,
    )(dis2d, brow, *agg)


def _tc_score(sagg, h, Pstack):
    C = len(sagg)
    F2 = 2 * C * 128
    NT = sagg[0].shape[0] // 128

    def body(*refs):
        p_ref = refs[0]
        tiles = refs[1:1 + 2 * C]
        score_ref = refs[1 + 2 * C]
        tile = jnp.concatenate([r[...] for r in tiles], axis=1)
        g = jnp.dot(tile, p_ref[...], preferred_element_type=jnp.float32)
        tr = jnp.swapaxes(g, 0, 1)
        score_ref[...] = tr[0:1, :].reshape(1, 1, 128)

    blk = pl.BlockSpec((128, 128), lambda i: (i, 0))
    return pl.pallas_call(
        body,
        grid=(NT,),
        in_specs=[pl.BlockSpec((F2, 128), lambda i: (0, 0))] + [blk] * (2 * C),
        out_specs=pl.BlockSpec((1, 1, 128), lambda i: (i, 0, 0)),
        out_shape=jax.ShapeDtypeStruct((NT, 1, 128), jnp.float32),
    )(Pstack, *sagg, *h)


def _tc_select(num, den, score2d, brel, batchf, act2d):
    NT = score2d.shape[0]
    NP = NT * 128
    NBITS = max(1, (NP - 1).bit_length())

    def body(sc_ref, br, bf_ref, act_ref, keep_ref, t_ref):
        score = sc_ref[...].reshape(NT, 128) + br[...]
        sbits = lax.bitcast_convert_type(score, jnp.int32)
        ubits = lax.bitcast_convert_type(score, jnp.uint32)
        key = jnp.where(sbits < 0, ~ubits, ubits | jnp.uint32(0x80000000))
        gi = (lax.broadcasted_iota(jnp.int32, (NT, 128), 0) * 128
              + lax.broadcasted_iota(jnp.int32, (NT, 128), 1))
        actm = act_ref[...].reshape(NT, 128) > 0.0
        bf = bf_ref[...].reshape(NT, 128)
        keep = jnp.zeros((NT, 128), jnp.float32)
        for g in range(NG):
            m = jnp.logical_and(bf == float(g), actm)
            c = jnp.sum(m.astype(jnp.int32))
            k = (num * c + (den - 1)) // den

            def tbody(it, T):
                bit = lax.shift_left(jnp.uint32(1), (31 - it).astype(jnp.uint32))
                Tc = T | bit
                cnt = jnp.sum(jnp.where(jnp.logical_and(m, key >= Tc), 1, 0))
                return jnp.where(cnt >= k, Tc, T)

            T = lax.fori_loop(0, 32, tbody, jnp.uint32(0))
            cgt = jnp.sum(jnp.where(jnp.logical_and(m, key > T), 1, 0))
            q = k - cgt
            meq = jnp.logical_and(m, key == T)

            def ebody(it, Ei):
                bit = lax.shift_left(jnp.int32(1), (NBITS - 1 - it).astype(jnp.int32))
                Ec = Ei | bit
                ce = jnp.sum(jnp.where(jnp.logical_and(meq, gi < Ec), 1, 0))
                return jnp.where(ce < q, Ec, Ei)

            E = lax.fori_loop(0, NBITS, ebody, jnp.int32(0))
            kg = jnp.logical_and(
                m,
                jnp.logical_or(
                    key > T,
                    jnp.logical_and(jnp.logical_and(meq, gi <= E), q > 0)))
            keep = keep + kg.astype(jnp.float32)
        keep_ref[...] = keep.reshape(NT, 1, 128)
        t_ref[...] = (jnp.tanh(score) * keep).reshape(NT, 1, 128)

    return pl.pallas_call(
        body,
        out_shape=[jax.ShapeDtypeStruct((NT, 1, 128), jnp.float32)] * 2,
    )(score2d, brel, batchf, act2d)


def _tc_final(dis2d, b4row, batchf, keepf, Wlp, blr, a0, a1):
    NT = dis2d.shape[0]
    F = 256

    def body(dis_ref, b_ref, bf_ref, k_ref, w_ref, bl_ref, a0_ref, a1_ref,
             out_ref, maxs, sums, cnts):
        i = pl.program_id(0)

        @pl.when(i == 0)
        def _init():
            maxs[...] = jnp.full((NG, F), -jnp.inf, jnp.float32)
            sums[...] = jnp.zeros((NG, F), jnp.float32)
            cnts[...] = jnp.zeros((NG, 128), jnp.float32)

        Md = _colbc(dis_ref[...].reshape(1, 128))
        tile = jnp.concatenate([a0_ref[...] * Md, a1_ref[...] * Md], axis=1) + b_ref[...]
        bcol = _colbc(bf_ref[...].reshape(1, 128))[:, 0:1]
        kcol = _colbc(k_ref[...].reshape(1, 128))[:, 0:1]
        for g in range(NG):
            m = jnp.logical_and(bcol == float(g), kcol > 0.0)
            mv = jnp.where(m, tile, -jnp.inf)
            maxs[g:g + 1, :] = jnp.maximum(maxs[g:g + 1, :],
                                           jnp.max(mv, axis=0, keepdims=True))
            sv = jnp.where(m, tile, 0.0)
            sums[g:g + 1, :] = sums[g:g + 1, :] + jnp.sum(sv, axis=0, keepdims=True)
            cnts[g:g + 1, :] = cnts[g:g + 1, :] + jnp.sum(m.astype(jnp.float32))

        @pl.when(i == NT - 1)
        def _fin():
            cnt = cnts[...][:, 0:1]
            meanp = sums[...] / jnp.maximum(cnt, 1.0)
            feat = jnp.concatenate([maxs[...], meanp], axis=1)
            res = jnp.dot(feat, w_ref[...], preferred_element_type=jnp.float32)
            out_ref[...] = res[:, 0:1] + bl_ref[...]

    row = pl.BlockSpec((1, 1, 128), lambda i: (i, 0, 0))
    blk = pl.BlockSpec((128, 128), lambda i: (i, 0))
    return pl.pallas_call(
        body,
        grid=(NT,),
        in_specs=[row,
                  pl.BlockSpec((1, F), lambda i: (0, 0)),
                  row, row,
                  pl.BlockSpec((2 * F, 128), lambda i: (0, 0)),
                  pl.BlockSpec((1, 1), lambda i: (0, 0)),
                  blk, blk],
        out_specs=pl.BlockSpec((NG, 1), lambda i: (0, 0)),
        out_shape=jax.ShapeDtypeStruct((NG, 1), jnp.float32),
        scratch_shapes=[pltpu.VMEM((NG, F), jnp.float32),
                        pltpu.VMEM((NG, F), jnp.float32),
                        pltpu.VMEM((NG, 128), jnp.float32)],
    )(dis2d, b4row, batchf, keepf, Wlp, blr, a0, a1)


_SC_MESH = dict(core_axis_name="c", subcore_axis_name="s")
_SC_PARAMS = pltpu.CompilerParams(use_tc_tiling_on_sc=False,
                                  needs_layout_passes=False)


def _sc_edge_update(srcr, dstr, keepflat, zrow, n_zero):
    ER, _ = srcr.shape
    NP = keepflat.shape[0]
    SL = NP // 16
    RW = ER // 32
    mesh = plsc.VectorSubcoreMesh(**_SC_MESH)

    def body(src_h, dst_h, keep_h, z_h, newsrc_h, deg_h,
             keep_vm, sall, dall, vv, ns, acc, semS):
        cid = lax.axis_index("c")
        sid = lax.axis_index("s")
        wid = sid * 2 + cid
        rbase = wid * RW
        pltpu.sync_copy(z_h, acc.at[pl.ds(sid * SL, SL)])
        pltpu.sync_copy(keep_h, keep_vm)
        pltpu.sync_copy(src_h.at[pl.ds(rbase, RW)], sall)
        pltpu.sync_copy(dst_h.at[pl.ds(rbase, RW)], dall)
        plsc.subcore_barrier()

        def rowloop(r, carry):
            for w in range(8):
                sl = pl.ds(w * 16, 16)
                sv = sall[r, sl]
                dv = dall[r, sl]
                k1 = plsc.load_gather(keep_vm, [sv])
                k2 = plsc.load_gather(keep_vm, [dv])
                val = k1 * k2
                vv[r, sl] = val
                ns[r, sl] = jnp.where(val > 0.0, sv,
                                      jnp.full((16,), n_zero, jnp.int32))
            return carry

        lax.fori_loop(0, RW, rowloop, 0)
        pltpu.sync_copy(ns, newsrc_h.at[pl.ds(rbase, RW)])

        def fire(r, carry):
            pltpu.async_copy(vv.at[r], acc.at[dall.at[r]], semS, add=True)
            return carry

        lax.fori_loop(0, RW, fire, 0)

        def drain(r, carry):
            pltpu.make_async_copy(vv.at[0], acc.at[dall.at[0]], semS).wait()
            return carry

        lax.fori_loop(0, RW, drain, 0)
        plsc.subcore_barrier()
        pltpu.sync_copy(acc.at[pl.ds(sid * SL, SL)],
                        deg_h.at[pl.ds(cid * NP + sid * SL, SL)])

    f = pl.kernel(
        body,
        out_type=[jax.ShapeDtypeStruct((ER, 128), jnp.int32),
                  jax.ShapeDtypeStruct((2 * NP,), jnp.float32)],
        mesh=mesh,
        compiler_params=_SC_PARAMS,
        scratch_types=[
            pltpu.VMEM((NP,), jnp.float32),
            pltpu.VMEM((RW, 128), jnp.int32),
            pltpu.VMEM((RW, 128), jnp.int32),
            pltpu.VMEM((RW, 128), jnp.float32),
            pltpu.VMEM((RW, 128), jnp.int32),
            pltpu.VMEM_SHARED((NP,), jnp.float32),
            pltpu.SemaphoreType.DMA,
        ],
    )
    return f(srcr, dstr, keepflat, zrow)


def _sc_row_scatter(ychunks, srcr, dstr, zrows):
    C = len(ychunks)
    ER, _ = srcr.shape
    NP = ychunks[0].shape[0]
    SLR = NP // 16
    RW = ER // 16
    BL = 8
    NBLK = RW // BL
    mesh = plsc.VectorSubcoreMesh(**_SC_MESH)

    def body(*refs):
        ys = refs[:C]
        src_h, dst_h, z_h = refs[C], refs[C + 1], refs[C + 2]
        aggs = refs[C + 3:C + 3 + C]
        dall, sblk, rows, acc, semG, semS0, semS1, semI = refs[C + 3 + C:]
        cid = lax.axis_index("c")
        sid = lax.axis_index("s")
        rbase = sid * RW
        pltpu.sync_copy(dst_h.at[pl.ds(rbase, RW)], dall)
        for chunk in range(C):
            @pl.when((chunk % 2) == cid)
            def _(chunk=chunk):
                yt = ys[chunk]
                pltpu.sync_copy(z_h, acc.at[pl.ds(sid * SLR, SLR)])
                plsc.subcore_barrier()

                def drain_g(bs):
                    pltpu.make_async_copy(
                        yt.at[dall.at[0]],
                        rows.at[pl.ds(bs * 128, 128)], semG).wait()

                def drain_s(bs, sem):
                    pltpu.make_async_copy(
                        rows.at[pl.ds(bs * 128, 128)],
                        acc.at[dall.at[0]], sem).wait()

                pltpu.sync_copy(src_h.at[pl.ds(rbase, BL)],
                                sblk.at[pl.ds(0, BL)])
                pltpu.async_copy(yt.at[sblk.at[0]],
                                 rows.at[pl.ds(0, 128)], semG)

                def block(bb, carry):
                    half = (bb % 2) * BL

                    @pl.when(bb + 1 < NBLK)
                    def _pf():
                        nh = ((bb + 1) % 2) * BL
                        pltpu.async_copy(
                            src_h.at[pl.ds(rbase + (bb + 1) * BL, BL)],
                            sblk.at[pl.ds(nh, BL)], semI)

                    for j in range(BL):
                        g = bb * BL + j
                        bs = j % 2
                        sem_cur = semS0 if bs == 0 else semS1
                        sem_oth = semS1 if bs == 0 else semS0
                        drain_g(bs)
                        pltpu.async_copy(rows.at[pl.ds(bs * 128, 128)],
                                         acc.at[dall.at[g]], sem_cur, add=True)

                        @pl.when(g >= 1)
                        def _ds(bs=bs, sem_oth=sem_oth):
                            drain_s(1 - bs, sem_oth)

                        if j < BL - 1:
                            pltpu.async_copy(yt.at[sblk.at[half + j + 1]],
                                             rows.at[pl.ds((1 - bs) * 128, 128)],
                                             semG)
                        else:
                            @pl.when(bb + 1 < NBLK)
                            def _fn(bs=bs, bb=bb):
                                nh = ((bb + 1) % 2) * BL
                                pltpu.make_async_copy(
                                    src_h.at[pl.ds(rbase, BL)],
                                    sblk.at[pl.ds(nh, BL)], semI).wait()
                                pltpu.async_copy(yt.at[sblk.at[nh]],
                                                 rows.at[pl.ds((1 - bs) * 128, 128)],
                                                 semG)
                    return carry

                lax.fori_loop(0, NBLK, block, 0)
                drain_s(1, semS1)
                plsc.subcore_barrier()
                pltpu.sync_copy(acc.at[pl.ds(sid * SLR, SLR)],
                                aggs[chunk].at[pl.ds(sid * SLR, SLR)])

    f = pl.kernel(
        body,
        out_type=[jax.ShapeDtypeStruct((NP, 128), jnp.float32)] * C,
        mesh=mesh,
        compiler_params=_SC_PARAMS,
        scratch_types=[
            pltpu.VMEM((RW, 128), jnp.int32),
            pltpu.VMEM((2 * BL, 128), jnp.int32),
            pltpu.VMEM((256, 128), jnp.float32),
            pltpu.VMEM_SHARED((NP, 128), jnp.float32),
            pltpu.SemaphoreType.DMA,
            pltpu.SemaphoreType.DMA,
            pltpu.SemaphoreType.DMA,
            pltpu.SemaphoreType.DMA,
        ],
    )
    outs = f(*ychunks, srcr, dstr, zrows)
    return list(outs) if isinstance(outs, (list, tuple)) else [outs]


def kernel(x, edge_index, batch, W1, b1, W2, b2, W3, b3, W4, b4,
           P1_rel, P1_root, P1_b, P2_rel, P2_root, P2_b,
           P3_rel, P3_root, P3_b, Wl, bl):
    N, F0 = x.shape
    E = edge_index.shape[1]
    NP = ((N + 1 + 2047) // 2048) * 2048
    NT = NP // 128
    EP = ((E + 16383) // 16384) * 16384

    f32 = jnp.float32

    xp = jnp.pad(x, ((0, NP - N), (0, 0)))
    xch = [xp[:, c * 128:(c + 1) * 128] for c in range(F0 // 128)]
    src0 = jnp.concatenate(
        [edge_index[0].astype(jnp.int32),
         jnp.full((EP - E,), N, jnp.int32)]).reshape(EP // 128, 128)
    dstr = jnp.concatenate(
        [edge_index[1].astype(jnp.int32),
         jnp.zeros((EP - E,), jnp.int32)]).reshape(EP // 128, 128)
    batchf = jnp.concatenate(
        [batch.astype(f32), jnp.full((NP - N,), float(NG), f32)]).reshape(NT, 1, 128)
    act0_flat = (jnp.arange(NP) < N).astype(f32)
    act0 = act0_flat.reshape(NT, 1, 128)
    ones2d = jnp.ones((NT, 1, 128), f32)
    zrow = jnp.zeros((NP // 16,), f32)
    zrows = jnp.zeros((NP // 16, 128), f32)

    def pstack(Pr, Pt):
        ps = jnp.concatenate([Pr, Pt], axis=0)
        return jnp.concatenate([ps, jnp.zeros((ps.shape[0], 127), f32)], axis=1)

    Pst1 = pstack(P1_rel, P1_root)
    Pst2 = pstack(P2_rel, P2_root)
    Pst3 = pstack(P3_rel, P3_root)
    Wlp = jnp.concatenate([Wl, jnp.zeros((Wl.shape[0], 127), f32)], axis=1)
    b1r, b2r = b1.reshape(1, -1), b2.reshape(1, -1)
    b3r, b4r = b3.reshape(1, -1), b4.reshape(1, -1)
    blr = bl.reshape(1, 1)

    def parts(flat):
        return flat[:NP].reshape(NT, 1, 128), flat[NP:].reshape(NT, 1, 128)

    src1, deg1 = _sc_edge_update(src0, dstr, act0_flat, zrow, N)
    y1, dis1 = _tc_prep(*parts(deg1), ones2d, xch, W1, 4)
    agg1 = _sc_row_scatter(y1, src1, dstr, zrows)
    h1 = _tc_hpost(dis1, b1r, agg1, N)
    sagg1 = _sc_row_scatter(h1, src1, dstr, zrows)
    score1 = _tc_score(sagg1, h1, Pst1)
    keep1, t1 = _tc_select(3, 5, score1, P1_b.reshape(1, 1), batchf, act0)

    src2, deg2 = _sc_edge_update(src1, dstr, keep1.reshape(NP), zrow, N)
    y2, dis2 = _tc_prep(*parts(deg2), t1, h1, W2, 4)
    agg2 = _sc_row_scatter(y2, src2, dstr, zrows)
    h2 = _tc_hpost(dis2, b2r, agg2, N)
    sagg2 = _sc_row_scatter(h2, src2, dstr, zrows)
    score2 = _tc_score(sagg2, h2, Pst2)
    keep2, t2 = _tc_select(3, 5, score2, P2_b.reshape(1, 1), batchf, keep1)

    src3, deg3 = _sc_edge_update(src2, dstr, keep2.reshape(NP), zrow, N)
    y3, dis3 = _tc_prep(*parts(deg3), t2, h2, W3, 2)
    agg3 = _sc_row_scatter(y3, src3, dstr, zrows)
    h3 = _tc_hpost(dis3, b3r, agg3, N)
    sagg3 = _sc_row_scatter(h3, src3, dstr, zrows)
    score3 = _tc_score(sagg3, h3, Pst3)
    keep3, t3 = _tc_select(1, 2, score3, P3_b.reshape(1, 1), batchf, keep2)

    src4, deg4 = _sc_edge_update(src3, dstr, keep3.reshape(NP), zrow, N)
    y4, dis4 = _tc_prep(*parts(deg4), t3, h3, W4, 2)
    agg4 = _sc_row_scatter(y4, src4, dstr, zrows)
    return _tc_final(dis4, b4r, batchf, keep3, Wlp, blr, agg4[0], agg4[1])
